# Initial kernel scaffold; baseline (speedup 1.0000x reference)
import jax
import jax.numpy as jnp
from jax.experimental import pallas as pl


def kernel(x, edge_index, enc_W, enc_b, dec_W, dec_b, conv_Wl, conv_bl, conv_Wr, gg_Wl, gg_bl, gg_Wr):
    raise NotImplementedError("write your pallas kernel here")


# speedup vs baseline: 2.1278x; 2.1278x over previous
import functools

import jax
import jax.numpy as jnp
from jax import lax
from jax.experimental import pallas as pl
from jax.experimental.pallas import tpu as pltpu
from jax.experimental.pallas import tpu_sc as plsc

N = 10000
NPAD = 10240
E = 160000
EPAD = 163840
NCLASS = 40
F = 256
FC = 128
NSUB = 16
NCORE = 2
EPS = EPAD // NSUB
CH = 128
J = EPS // CH
BT = 640
RT = NPAD // NSUB
CW = 16


def _mesh():
    return plsc.VectorSubcoreMesh(
        core_axis_name="c", subcore_axis_name="s",
        num_cores=NCORE, num_subcores=NSUB)


@functools.lru_cache()
def _make_spmm(nf):
    @functools.partial(
        pl.kernel,
        out_type=jax.ShapeDtypeStruct((nf, NPAD, FC), jnp.float32),
        mesh=_mesh(),
        scratch_types=[
            pltpu.VMEM_SHARED((NPAD, FC), jnp.float32),
            pltpu.VMEM((J, CH), jnp.int32),
            pltpu.VMEM((J, CH), jnp.int32),
            pltpu.VMEM((J, CH), jnp.int32),
            pltpu.VMEM((CH, FC), jnp.float32),
            pltpu.SemaphoreType.DMA,
        ],
    )
    def spmm(data, gidx, sidx, zrows, out, acc, gi, ai, si, buf, sem):
        c = lax.axis_index("c")
        s = lax.axis_index("s")
        rbase = s * RT
        pltpu.sync_copy(gidx.at[s], gi)
        pltpu.sync_copy(sidx.at[s], si)
        for p in range(nf // 2):
            fc = c + 2 * p
            def adj(j, _):
                for t in range(CH // 16):
                    ai[j, pl.ds(t * 16, 16)] = gi[j, pl.ds(t * 16, 16)] * nf + fc
                return 0
            lax.fori_loop(0, J, adj, 0)
            pltpu.sync_copy(zrows, acc.at[pl.ds(rbase, RT)])
            plsc.subcore_barrier()
            def body(j, _):
                pltpu.async_copy(data.at[ai.at[j]], buf, sem).wait()
                pltpu.sync_copy(buf, acc.at[si.at[j]], add=True)
                return 0
            lax.fori_loop(0, J, body, 0)
            plsc.subcore_barrier()
            @pl.when(c == 0)
            def _():
                pltpu.sync_copy(acc.at[pl.ds(rbase, RT)],
                                out.at[2 * p, pl.ds(rbase, RT)])
            @pl.when(c == 1)
            def _():
                pltpu.sync_copy(acc.at[pl.ds(rbase, RT)],
                                out.at[2 * p + 1, pl.ds(rbase, RT)])
            if p + 1 < nf // 2:
                plsc.subcore_barrier()

    return spmm


@functools.lru_cache()
def _make_counts():
    @functools.partial(
        pl.kernel,
        out_type=jax.ShapeDtypeStruct((NCORE, NPAD, FC), jnp.float32),
        mesh=_mesh(),
        scratch_types=[
            pltpu.VMEM_SHARED((NPAD, FC), jnp.float32),
            pltpu.VMEM((J, CH), jnp.int32),
            pltpu.VMEM((CH, FC), jnp.float32),
        ],
    )
    def counts(eidx, zcnt, ones, out, acc, ii, ob):
        c = lax.axis_index("c")
        s = lax.axis_index("s")
        rbase = s * RT
        @pl.when(c == 0)
        def _():
            pltpu.sync_copy(eidx.at[s], ii)
        @pl.when(c == 1)
        def _():
            pltpu.sync_copy(eidx.at[NSUB + s], ii)
        pltpu.sync_copy(ones, ob)
        pltpu.sync_copy(zcnt, acc.at[pl.ds(rbase, RT)])
        plsc.subcore_barrier()
        def body(j, _):
            pltpu.sync_copy(ob, acc.at[ii.at[j]], add=True)
            return 0
        lax.fori_loop(0, J, body, 0)
        plsc.subcore_barrier()
        @pl.when(c == 0)
        def _():
            pltpu.sync_copy(acc.at[pl.ds(rbase, RT)],
                            out.at[0, pl.ds(rbase, RT)])
        @pl.when(c == 1)
        def _():
            pltpu.sync_copy(acc.at[pl.ds(rbase, RT)],
                            out.at[1, pl.ds(rbase, RT)])

    return counts


def _dg(a, w):
    return lax.dot_general(a, w, (((1,), (1,)), ((), ())),
                           preferred_element_type=jnp.float32)


def _enc_body(x_ref, w_ref, b_ref, o_ref):
    o_ref[...] = jnp.maximum(_dg(x_ref[...], w_ref[...]) + b_ref[...], 0.0)


def _enc(x, w, b):
    return pl.pallas_call(
        _enc_body,
        grid=(NPAD // BT,),
        in_specs=[
            pl.BlockSpec((BT, F), lambda i: (i, 0)),
            pl.BlockSpec((F, F), lambda i: (0, 0)),
            pl.BlockSpec((1, F), lambda i: (0, 0)),
        ],
        out_specs=pl.BlockSpec((BT, F), lambda i: (i, 0)),
        out_shape=jax.ShapeDtypeStruct((NPAD, F), jnp.float32),
    )(x, w, b)


def _dec_body(h_ref, w_ref, b_ref, o_ref):
    o_ref[...] = _dg(h_ref[...], w_ref[...]) + b_ref[...]


def _dec(h, w, b):
    return pl.pallas_call(
        _dec_body,
        grid=(NPAD // BT,),
        in_specs=[
            pl.BlockSpec((BT, F), lambda i: (i, 0)),
            pl.BlockSpec((FC, F), lambda i: (0, 0)),
            pl.BlockSpec((1, FC), lambda i: (0, 0)),
        ],
        out_specs=pl.BlockSpec((BT, FC), lambda i: (i, 0)),
        out_shape=jax.ShapeDtypeStruct((NPAD, FC), jnp.float32),
    )(h, w, b)


def _layer_a_body(h_ref, agg_ref, cnt_ref, wl_ref, bl_ref, wr_ref,
                  gwl_ref, gbl_ref, gwr_ref, xa_ref, y_ref):
    cnt = cnt_ref[:, 0:1]
    r = 1.0 / jnp.maximum(cnt, 1.0)
    agg = jnp.concatenate([agg_ref[0], agg_ref[1]], axis=1) * r
    h = h_ref[...]
    xa = jnp.maximum(_dg(agg, wl_ref[...]) + bl_ref[...]
                     + _dg(h, wr_ref[...]), 0.0)
    xx = jnp.maximum(_dg(agg, gwl_ref[...]) + gbl_ref[...]
                     + _dg(h, gwr_ref[...]), 0.0)
    xa_ref[...] = xa
    y_ref[...] = jnp.concatenate([xx, xx * xx], axis=1)


def _layer_a(h, aggs, cnt, wl, bl, wr, gwl, gbl, gwr):
    return pl.pallas_call(
        _layer_a_body,
        grid=(NPAD // BT,),
        in_specs=[
            pl.BlockSpec((BT, F), lambda i: (i, 0)),
            pl.BlockSpec((2, BT, FC), lambda i: (0, i, 0)),
            pl.BlockSpec((BT, FC), lambda i: (i, 0)),
            pl.BlockSpec((F, F), lambda i: (0, 0)),
            pl.BlockSpec((1, F), lambda i: (0, 0)),
            pl.BlockSpec((F, F), lambda i: (0, 0)),
            pl.BlockSpec((F, F), lambda i: (0, 0)),
            pl.BlockSpec((1, F), lambda i: (0, 0)),
            pl.BlockSpec((F, F), lambda i: (0, 0)),
        ],
        out_specs=[
            pl.BlockSpec((BT, F), lambda i: (i, 0)),
            pl.BlockSpec((BT, 2 * F), lambda i: (i, 0)),
        ],
        out_shape=[
            jax.ShapeDtypeStruct((NPAD, F), jnp.float32),
            jax.ShapeDtypeStruct((NPAD, 2 * F), jnp.float32),
        ],
    )(h, aggs, cnt, wl, bl, wr, gwl, gbl, gwr)


def _layer_b_body(h_ref, xa_ref, y_ref, ab_ref, cnt_ref, o_ref):
    cnt = cnt_ref[:, 0:1]
    r = 1.0 / jnp.maximum(cnt, 1.0)
    d = jnp.minimum(cnt, 1.0)
    xx = y_ref[...]
    a = jnp.concatenate([ab_ref[0], ab_ref[1]], axis=1) * r
    b = jnp.concatenate([ab_ref[2], ab_ref[3]], axis=1) * r
    tau = jnp.tanh((d * xx - 2.0 * a) * xx + b)
    h = h_ref[...]
    o_ref[...] = h + tau * (xa_ref[...] - h)


def _layer_b(h, xa, y, ab, cnt):
    return pl.pallas_call(
        _layer_b_body,
        grid=(NPAD // BT,),
        in_specs=[
            pl.BlockSpec((BT, F), lambda i: (i, 0)),
            pl.BlockSpec((BT, F), lambda i: (i, 0)),
            pl.BlockSpec((BT, F), lambda i: (i, 0)),
            pl.BlockSpec((4, BT, FC), lambda i: (0, i, 0)),
            pl.BlockSpec((BT, FC), lambda i: (i, 0)),
        ],
        out_specs=pl.BlockSpec((BT, F), lambda i: (i, 0)),
        out_shape=jax.ShapeDtypeStruct((NPAD, F), jnp.float32),
    )(h, xa, y, ab, cnt)


def kernel(x, edge_index, enc_W, enc_b, dec_W, dec_b,
           conv_Wl, conv_bl, conv_Wr, gg_Wl, gg_bl, gg_Wr):
    f32 = jnp.float32
    src = edge_index[0]
    dst = edge_index[1]
    pad = jnp.full((EPAD - E,), N, jnp.int32)
    srcp = jnp.concatenate([src, pad]).reshape(NSUB, J, CH)
    dstp = jnp.concatenate([dst, pad]).reshape(NSUB, J, CH)
    zrows = jnp.zeros((RT, FC), f32)
    ones = jnp.ones((CH, FC), f32)
    xp = jnp.pad(x, ((0, NPAD - N), (0, 0)))

    cnts = _make_counts()(jnp.concatenate([dstp, srcp], axis=0), zrows, ones)
    cnt_dst = cnts[0]
    cnt_src = cnts[1]
    spmm2 = _make_spmm(2)
    spmm4 = _make_spmm(4)

    blr = conv_bl.reshape(1, F)
    gblr = gg_bl.reshape(1, F)
    h = _enc(xp, enc_W, enc_b.reshape(1, F))
    for _ in range(4):
        aggs = spmm2(h.reshape(NPAD * 2, FC), srcp, dstp, zrows)
        xa, y = _layer_a(h, aggs, cnt_dst, conv_Wl, blr, conv_Wr,
                         gg_Wl, gblr, gg_Wr)
        ab = spmm4(y.reshape(NPAD * 4, FC), dstp, srcp, zrows)
        h = _layer_b(h, xa, y, ab, cnt_src)

    dw = jnp.pad(dec_W, ((0, FC - NCLASS), (0, 0)))
    db = jnp.pad(dec_b, (0, FC - NCLASS)).reshape(1, FC)
    out = _dec(h, dw, db)
    return out[:N, :NCLASS]
